# baseline (speedup 1.0000x reference)
import functools

import jax
import jax.numpy as jnp
import numpy as np
from jax import lax
from jax.experimental import pallas as pl
from jax.experimental.pallas import tpu as pltpu
from jax.experimental.pallas import tpu_sc as plsc

B = 1024
L = 150
D = 25
V = 5001
H = 500
C = 2
DH = 12
LP = 160
LG = LP // 16
BCT = 16

NC = 2
NS = 16
NW = NC * NS
BPW = B // NW
WCR = LP * DH

_P = np.zeros((D, DH), np.float32)
for _j in range(DH):
    _P[2 * _j, _j] = 0.5
    _P[2 * _j + 1, _j] = 0.5


def _precompute(emb, p_mat, w1, w2, b1r, b2r):
    def body(emb_ref, p_ref, w1_ref, w2_ref, b1_ref, b2_ref,
             pt_ref, wc_ref, bc_ref):
        pt_ref[...] = jnp.dot(emb_ref[...], p_ref[...],
                              preferred_element_type=jnp.float32)
        wc_ref[...] = jnp.dot(w1_ref[...], w2_ref[...],
                              preferred_element_type=jnp.float32)
        bcd = jnp.dot(b1_ref[...], w2_ref[...],
                      preferred_element_type=jnp.float32) + b2_ref[...]
        bc_ref[...] = jnp.concatenate([bcd] * (BCT // C), axis=1)

    return pl.pallas_call(
        body,
        out_shape=[
            jax.ShapeDtypeStruct((V, DH), jnp.float32),
            jax.ShapeDtypeStruct((L * DH, C), jnp.float32),
            jax.ShapeDtypeStruct((1, BCT), jnp.float32),
        ],
    )(emb, p_mat, w1, w2, b1r, b2r)


def _sc_fused(ptable, idxp, wc12, bct):
    mesh = plsc.VectorSubcoreMesh(core_axis_name="c", subcore_axis_name="s")

    @functools.partial(
        pl.kernel,
        mesh=mesh,
        out_type=jax.ShapeDtypeStruct((B * C,), jnp.float32),
        scratch_types=[
            pltpu.VMEM((V, DH), jnp.float32),
            pltpu.VMEM((BPW, LP), jnp.int32),
            pltpu.VMEM((WCR, C), jnp.float32),
            pltpu.VMEM((C, DH, LP), jnp.float32),
            pltpu.VMEM((1, BCT), jnp.float32),
            pltpu.VMEM((BPW, C, 16), jnp.float32),
            pltpu.VMEM((BPW * C,), jnp.float32),
            pltpu.SemaphoreType.DMA,
        ],
        compiler_params=pltpu.CompilerParams(
            use_tc_tiling_on_sc=False, needs_layout_passes=False),
    )
    def k(pt_hbm, idx_hbm, wc_hbm, bc_hbm, out_hbm,
          pt_v, idx_v, wcr_v, wct_v, bc_v, acc_v, out_v, sem):
        wid = lax.axis_index("s") * NC + lax.axis_index("c")
        table_cp = pltpu.make_async_copy(pt_hbm, pt_v, sem)
        table_cp.start()
        pltpu.sync_copy(idx_hbm.at[wid], idx_v)
        pltpu.sync_copy(wc_hbm, wcr_v.at[pl.ds(0, L * DH)])
        pltpu.sync_copy(bc_hbm, bc_v)

        lanes = lax.iota(jnp.int32, 16)
        zero16 = jnp.zeros((16,), jnp.float32)

        def build_wct(lg, carry):
            lvec = lg * 16 + lanes
            rows = lvec * DH
            valid = lvec < L
            for c in range(C):
                cols = jnp.full((16,), c, jnp.int32)
                for j in range(DH):
                    vec = plsc.load_gather(wcr_v, [rows + j, cols])
                    wct_v[c, j, pl.ds(lg * 16, 16)] = jnp.where(
                        valid, vec, 0.0)
            return carry

        lax.fori_loop(0, LG, build_wct, 0)
        bc_tiled = bc_v[0]

        def zero_acc(b, carry):
            acc_v[b, 0] = zero16
            acc_v[b, 1] = zero16
            return carry

        lax.fori_loop(0, BPW, zero_acc, 0)

        table_cp.wait()

        def lg_body(lg, carry):
            w0s = [wct_v[0, j, pl.ds(lg * 16, 16)] for j in range(DH)]
            w1s = [wct_v[1, j, pl.ds(lg * 16, 16)] for j in range(DH)]

            def b_body(b, carry2):
                idx16 = idx_v[b, pl.ds(lg * 16, 16)]
                a0 = acc_v[b, 0]
                a1 = acc_v[b, 1]
                for j in range(DH):
                    cols = jnp.full((16,), j, jnp.int32)
                    row = plsc.load_gather(pt_v, [idx16, cols])
                    a0 = a0 + row * w0s[j]
                    a1 = a1 + row * w1s[j]
                acc_v[b, 0] = a0
                acc_v[b, 1] = a1
                return carry2

            lax.fori_loop(0, BPW, b_body, 0)
            return carry

        lax.fori_loop(0, LG, lg_body, 0)

        def assemble(g, carry):
            out_vec = zero16
            for kk in range(8):
                s0 = jnp.sum(acc_v[g * 8 + kk, 0])
                s1 = jnp.sum(acc_v[g * 8 + kk, 1])
                out_vec = jnp.where(lanes == 2 * kk, s0, out_vec)
                out_vec = jnp.where(lanes == 2 * kk + 1, s1, out_vec)
            out_v[pl.ds(g * 16, 16)] = out_vec + bc_tiled
            return carry

        lax.fori_loop(0, BPW // 8, assemble, 0)
        pltpu.sync_copy(out_v, out_hbm.at[pl.ds(wid * (BPW * C), BPW * C)])

    return k(ptable, idxp, wc12, bct)


def kernel(sents, _, emb_table, W1, b1, W2, b2):
    ptable, wc12, bct = _precompute(
        emb_table, jnp.asarray(_P), W1, W2,
        b1.reshape(1, H), b2.reshape(1, C))
    idxp = jnp.pad(sents.astype(jnp.int32), ((0, 0), (0, LP - L)))
    idxp = idxp.reshape(NW, BPW, LP)
    out = _sc_fused(ptable, idxp, wc12, bct)
    return out.reshape(B, C)


# speedup vs baseline: 1.0041x; 1.0041x over previous
import functools

import jax
import jax.numpy as jnp
import numpy as np
from jax import lax
from jax.experimental import pallas as pl
from jax.experimental.pallas import tpu as pltpu
from jax.experimental.pallas import tpu_sc as plsc

B = 1024
L = 150
D = 25
V = 5001
H = 500
C = 2
DH = 12
LP = 160
LG = LP // 16
BCT = 16

NC = 2
NS = 16
NW = NC * NS
BPW = B // NW
WCR = LP * DH

_P = np.zeros((D, DH), np.float32)
for _j in range(DH):
    _P[2 * _j, _j] = 0.5
    _P[2 * _j + 1, _j] = 0.5


def _precompute(emb, p_mat, w1, w2, b1r, b2r):
    def body(emb_ref, p_ref, w1_ref, w2_ref, b1_ref, b2_ref,
             pt_ref, wc_ref, bc_ref):
        pt_ref[...] = jnp.dot(emb_ref[...], p_ref[...],
                              preferred_element_type=jnp.float32)
        wc_ref[...] = jnp.dot(w1_ref[...], w2_ref[...],
                              preferred_element_type=jnp.float32)
        bcd = jnp.dot(b1_ref[...], w2_ref[...],
                      preferred_element_type=jnp.float32) + b2_ref[...]
        bc_ref[...] = jnp.concatenate([bcd] * (BCT // C), axis=1)

    return pl.pallas_call(
        body,
        out_shape=[
            jax.ShapeDtypeStruct((V, DH), jnp.float32),
            jax.ShapeDtypeStruct((L * DH, C), jnp.float32),
            jax.ShapeDtypeStruct((1, BCT), jnp.float32),
        ],
    )(emb, p_mat, w1, w2, b1r, b2r)


def _sc_fused(ptable, idxp, wc12, bct):
    mesh = plsc.VectorSubcoreMesh(core_axis_name="c", subcore_axis_name="s")

    @functools.partial(
        pl.kernel,
        mesh=mesh,
        out_type=jax.ShapeDtypeStruct((B * C,), jnp.float32),
        scratch_types=[
            pltpu.VMEM((V, DH), jnp.float32),
            pltpu.VMEM((BPW, LP), jnp.int32),
            pltpu.VMEM((WCR, C), jnp.float32),
            pltpu.VMEM((C, DH, LP), jnp.float32),
            pltpu.VMEM((1, BCT), jnp.float32),
            pltpu.VMEM((BPW, C, 16), jnp.float32),
            pltpu.VMEM((BPW * C,), jnp.float32),
            pltpu.SemaphoreType.DMA,
        ],
        compiler_params=pltpu.CompilerParams(
            use_tc_tiling_on_sc=False, needs_layout_passes=False),
    )
    def k(pt_hbm, idx_hbm, wc_hbm, bc_hbm, out_hbm,
          pt_v, idx_v, wcr_v, wct_v, bc_v, acc_v, out_v, sem):
        wid = lax.axis_index("s") * NC + lax.axis_index("c")
        table_cp = pltpu.make_async_copy(pt_hbm, pt_v, sem)
        table_cp.start()
        pltpu.sync_copy(idx_hbm.at[wid], idx_v)
        pltpu.sync_copy(wc_hbm, wcr_v.at[pl.ds(0, L * DH)])
        pltpu.sync_copy(bc_hbm, bc_v)

        lanes = lax.iota(jnp.int32, 16)
        zero16 = jnp.zeros((16,), jnp.float32)

        def build_wct(lg, carry):
            lvec = lg * 16 + lanes
            rows = lvec * DH
            valid = lvec < L
            for c in range(C):
                cols = jnp.full((16,), c, jnp.int32)
                for j in range(DH):
                    vec = plsc.load_gather(wcr_v, [rows + j, cols])
                    wct_v[c, j, pl.ds(lg * 16, 16)] = jnp.where(
                        valid, vec, 0.0)
            return carry

        lax.fori_loop(0, LG, build_wct, 0)
        bc_tiled = bc_v[0]

        def zero_acc(b, carry):
            acc_v[b, 0] = zero16
            acc_v[b, 1] = zero16
            return carry

        lax.fori_loop(0, BPW, zero_acc, 0)

        table_cp.wait()

        def lg_body(lg, carry):
            w0s = [wct_v[0, j, pl.ds(lg * 16, 16)] for j in range(DH)]
            w1s = [wct_v[1, j, pl.ds(lg * 16, 16)] for j in range(DH)]

            def b_body(b, carry2):
                idx16 = idx_v[b, pl.ds(lg * 16, 16)]
                rows = [plsc.load_gather(pt_v,
                                         [idx16, jnp.full((16,), j, jnp.int32)])
                        for j in range(DH)]
                t0 = [rows[j] * w0s[j] for j in range(DH)]
                t1 = [rows[j] * w1s[j] for j in range(DH)]
                while len(t0) > 1:
                    t0 = [t0[i] + t0[i + 1] for i in range(0, len(t0) - 1, 2)] \
                        + ([t0[-1]] if len(t0) % 2 else [])
                    t1 = [t1[i] + t1[i + 1] for i in range(0, len(t1) - 1, 2)] \
                        + ([t1[-1]] if len(t1) % 2 else [])
                acc_v[b, 0] = acc_v[b, 0] + t0[0]
                acc_v[b, 1] = acc_v[b, 1] + t1[0]
                return carry2

            lax.fori_loop(0, BPW, b_body, 0)
            return carry

        lax.fori_loop(0, LG, lg_body, 0)

        def assemble(g, carry):
            out_vec = zero16
            for kk in range(8):
                s0 = jnp.sum(acc_v[g * 8 + kk, 0])
                s1 = jnp.sum(acc_v[g * 8 + kk, 1])
                out_vec = jnp.where(lanes == 2 * kk, s0, out_vec)
                out_vec = jnp.where(lanes == 2 * kk + 1, s1, out_vec)
            out_v[pl.ds(g * 16, 16)] = out_vec + bc_tiled
            return carry

        lax.fori_loop(0, BPW // 8, assemble, 0)
        pltpu.sync_copy(out_v, out_hbm.at[pl.ds(wid * (BPW * C), BPW * C)])

    return k(ptable, idxp, wc12, bct)


def kernel(sents, _, emb_table, W1, b1, W2, b2):
    ptable, wc12, bct = _precompute(
        emb_table, jnp.asarray(_P), W1, W2,
        b1.reshape(1, H), b2.reshape(1, C))
    idxp = jnp.pad(sents.astype(jnp.int32), ((0, 0), (0, LP - L)))
    idxp = idxp.reshape(NW, BPW, LP)
    out = _sc_fused(ptable, idxp, wc12, bct)
    return out.reshape(B, C)


# speedup vs baseline: 1.3626x; 1.3571x over previous
import functools

import jax
import jax.numpy as jnp
import numpy as np
from jax import lax
from jax.experimental import pallas as pl
from jax.experimental.pallas import tpu as pltpu
from jax.experimental.pallas import tpu_sc as plsc

B = 1024
L = 150
D = 25
V = 5001
H = 500
C = 2
DH = 12
DPAD = 16
NIDX = B * L

NC = 2
NS = 16
NW = NC * NS
B_PER_W = NIDX // NW
CH = 40
CW = 120
BPW = B // NW
GRP = 8
NG = BPW // GRP
CPG = CH // NG
WCR = L * DH + 8

_P = np.zeros((D, DPAD), np.float32)
for _j in range(DH):
    _P[2 * _j, _j] = 0.5
    _P[2 * _j + 1, _j] = 0.5


def _precompute(emb, p_mat, w1, w2, b1r, b2r):
    def body(emb_ref, p_ref, w1_ref, w2_ref, b1_ref, b2_ref,
             pt_ref, wc_ref, bc_ref):
        pt_ref[...] = jnp.dot(emb_ref[...], p_ref[...],
                              preferred_element_type=jnp.float32)
        wc_ref[...] = jnp.dot(w1_ref[...], w2_ref[...],
                              preferred_element_type=jnp.float32)
        bcd = jnp.dot(b1_ref[...], w2_ref[...],
                      preferred_element_type=jnp.float32) + b2_ref[...]
        bc_ref[...] = jnp.concatenate([bcd] * (DPAD // C), axis=1)

    return pl.pallas_call(
        body,
        out_shape=[
            jax.ShapeDtypeStruct((V, DPAD), jnp.float32),
            jax.ShapeDtypeStruct((L * DH, C), jnp.float32),
            jax.ShapeDtypeStruct((1, DPAD), jnp.float32),
        ],
    )(emb, p_mat, w1, w2, b1r, b2r)


def _sc_fused(ptable, idx3, wc12, bct):
    mesh = plsc.VectorSubcoreMesh(core_axis_name="c", subcore_axis_name="s")

    @functools.partial(
        pl.kernel,
        mesh=mesh,
        out_type=jax.ShapeDtypeStruct((B * C,), jnp.float32),
        scratch_types=[
            pltpu.VMEM((CH, CW), jnp.int32),
            pltpu.VMEM((B_PER_W, DPAD), jnp.float32),
            pltpu.VMEM((WCR, C), jnp.float32),
            pltpu.VMEM((L, C, DPAD), jnp.float32),
            pltpu.VMEM((1, DPAD), jnp.float32),
            pltpu.VMEM((BPW * C,), jnp.float32),
            pltpu.SemaphoreType.DMA,
        ],
        compiler_params=pltpu.CompilerParams(
            use_tc_tiling_on_sc=False, needs_layout_passes=False),
    )
    def k(pt_hbm, idx_hbm, wc_hbm, bc_hbm, out_hbm,
          idx_v, rows_v, wcr_v, wc_v, bc_v, out_v, sem):
        wid = lax.axis_index("s") * NC + lax.axis_index("c")
        pltpu.sync_copy(idx_hbm.at[wid], idx_v)
        pltpu.sync_copy(wc_hbm, wcr_v.at[pl.ds(0, L * DH)])
        pltpu.sync_copy(bc_hbm, bc_v)

        def fire(j, carry):
            pltpu.make_async_copy(
                pt_hbm.at[idx_v.at[j]],
                rows_v.at[pl.ds(j * CW, CW)], sem).start()
            return carry

        lax.fori_loop(0, CH, fire, 0)

        lanes = lax.iota(jnp.int32, 16)
        zero16 = jnp.zeros((16,), jnp.float32)

        def build_wc(l, carry):
            rows = l * DH + lanes
            for c in range(C):
                cols = jnp.full((16,), c, jnp.int32)
                vec = plsc.load_gather(wcr_v, [rows, cols])
                wc_v[l, c] = jnp.where(lanes < DH, vec, 0.0)
            return carry

        lax.fori_loop(0, L, build_wc, 0)
        bc_tiled = bc_v[0]

        def group_body(g, carry):
            def drain(i, carry2):
                j = g * CPG + i
                pltpu.make_async_copy(
                    pt_hbm.at[idx_v.at[j]],
                    rows_v.at[pl.ds(j * CW, CW)], sem).wait()
                return carry2

            lax.fori_loop(0, CPG, drain, 0)

            def l_body(l, accs):
                w0 = wc_v[l, 0]
                w1 = wc_v[l, 1]
                base = g * (GRP * L) + l
                new = []
                for kk in range(GRP):
                    row = rows_v[base + kk * L]
                    new.append(accs[2 * kk] + row * w0)
                    new.append(accs[2 * kk + 1] + row * w1)
                return tuple(new)

            accs = lax.fori_loop(0, L, l_body, (zero16,) * (2 * GRP))
            out_vec = zero16
            for kk in range(GRP):
                s0 = jnp.sum(accs[2 * kk])
                s1 = jnp.sum(accs[2 * kk + 1])
                out_vec = jnp.where(lanes == 2 * kk, s0, out_vec)
                out_vec = jnp.where(lanes == 2 * kk + 1, s1, out_vec)
            out_v[pl.ds(g * 16, 16)] = out_vec + bc_tiled
            return carry

        lax.fori_loop(0, NG, group_body, 0)
        pltpu.sync_copy(out_v, out_hbm.at[pl.ds(wid * (BPW * C), BPW * C)])

    return k(ptable, idx3, wc12, bct)


def kernel(sents, _, emb_table, W1, b1, W2, b2):
    ptable, wc12, bct = _precompute(
        emb_table, jnp.asarray(_P), W1, W2,
        b1.reshape(1, H), b2.reshape(1, C))
    idx3 = sents.astype(jnp.int32).reshape(NW, CH, CW)
    out = _sc_fused(ptable, idx3, wc12, bct)
    return out.reshape(B, C)
